# Initial kernel scaffold; baseline (speedup 1.0000x reference)
import jax
import jax.numpy as jnp
from jax.experimental import pallas as pl


def kernel(x, norm_scale, gate_w, gate_b, mlp1_weight, mlp1_bias, mlp2_weight, mlp2_bias):
    raise NotImplementedError("write your pallas kernel here")


# speedup vs baseline: 2.9275x; 2.9275x over previous
import functools
import jax
import jax.numpy as jnp
from jax import lax
from jax.experimental import pallas as pl
from jax.experimental.pallas import tpu as pltpu

H = 1024
I = 1024
E = 8
K = 2
B = 32
LIMIT = 7.0
EPS = 1e-5


def _moe_kernel(x_ref, ns_ref, gw_ref, gb_ref, w1_ref, b1_ref, w2_ref, b2_ref,
                out_ref, t_s, r_s, acc_s):
    e = pl.program_id(0)

    @pl.when(e == 0)
    def _routing():
        xf = x_ref[...].astype(jnp.float32)
        ms = jnp.mean(xf * xf, axis=1, keepdims=True)
        t = xf * lax.rsqrt(ms + EPS) * ns_ref[...]
        t_bf = t.astype(jnp.bfloat16)
        t_s[...] = t_bf
        g = lax.dot_general(t_bf, gw_ref[...], (((1,), (1,)), ((), ())),
                            preferred_element_type=jnp.float32)
        gf = g + gb_ref[...].astype(jnp.float32)
        ids = lax.broadcasted_iota(jnp.int32, (B, E), 1)
        m1 = jnp.max(gf, axis=1, keepdims=True)
        i1 = jnp.min(jnp.where(gf == m1, ids, E), axis=1, keepdims=True)
        g2 = jnp.where(ids == i1, -jnp.inf, gf)
        m2 = jnp.max(g2, axis=1, keepdims=True)
        i2 = jnp.min(jnp.where(g2 == m2, ids, E), axis=1, keepdims=True)
        m1 = m1.astype(jnp.bfloat16).astype(jnp.float32)
        m2 = m2.astype(jnp.bfloat16).astype(jnp.float32)
        d = jnp.exp(m2 - m1)
        p1 = 1.0 / (1.0 + d)
        p2 = d / (1.0 + d)
        r_s[:, 0:1] = i1.astype(jnp.float32)
        r_s[:, 1:2] = i2.astype(jnp.float32)
        r_s[:, 2:3] = p1
        r_s[:, 3:4] = p2

    t_bf = t_s[...]
    w1 = w1_ref[0]
    h = lax.dot_general(t_bf, w1, (((1,), (1,)), ((), ())),
                        preferred_element_type=jnp.float32)
    h_bf = h.astype(jnp.bfloat16) + b1_ref[0]
    hf = h_bf.astype(jnp.float32)
    h3 = hf.reshape(B, I, 2)
    x_glu = h3[:, :, 0]
    x_lin = h3[:, :, 1]
    x_glu = jnp.minimum(x_glu, LIMIT)
    x_lin = jnp.clip(x_lin, -LIMIT, LIMIT)
    out_glu = x_glu * (1.0 / (1.0 + jnp.exp(-1.702 * x_glu)))
    hmid = (out_glu * (x_lin + 1.0)).astype(jnp.bfloat16)

    w2 = w2_ref[0]
    y = lax.dot_general(hmid, w2, (((1,), (1,)), ((), ())),
                        preferred_element_type=jnp.float32)
    y_bf = y.astype(jnp.bfloat16) + b2_ref[0]

    ef = e.astype(jnp.float32)
    w_e = (jnp.where(r_s[:, 0:1] == ef, r_s[:, 2:3], 0.0)
           + jnp.where(r_s[:, 1:2] == ef, r_s[:, 3:4], 0.0))
    contrib = w_e * y_bf.astype(jnp.float32)

    @pl.when(e == 0)
    def _init():
        acc_s[...] = contrib

    @pl.when(e > 0)
    def _accum():
        acc_s[...] += contrib

    @pl.when(e == E - 1)
    def _finish():
        out_ref[...] = (x_ref[...].astype(jnp.float32) + acc_s[...]).astype(
            jnp.bfloat16)


@jax.jit
def kernel(x, norm_scale, gate_w, gate_b, mlp1_weight, mlp1_bias, mlp2_weight,
           mlp2_bias):
    ns = norm_scale.reshape(1, H).astype(jnp.float32)
    gb = gate_b.reshape(1, E)
    return pl.pallas_call(
        _moe_kernel,
        grid=(E,),
        in_specs=[
            pl.BlockSpec((B, H), lambda e: (0, 0)),
            pl.BlockSpec((1, H), lambda e: (0, 0)),
            pl.BlockSpec((E, H), lambda e: (0, 0)),
            pl.BlockSpec((1, E), lambda e: (0, 0)),
            pl.BlockSpec((1, 2 * I, H), lambda e: (e, 0, 0)),
            pl.BlockSpec((1, 1, 2 * I), lambda e: (e, 0, 0)),
            pl.BlockSpec((1, H, I), lambda e: (e, 0, 0)),
            pl.BlockSpec((1, 1, H), lambda e: (e, 0, 0)),
        ],
        out_specs=pl.BlockSpec((B, H), lambda e: (0, 0)),
        out_shape=jax.ShapeDtypeStruct((B, H), jnp.bfloat16),
        scratch_shapes=[
            pltpu.VMEM((B, H), jnp.bfloat16),
            pltpu.VMEM((B, 4), jnp.float32),
            pltpu.VMEM((B, H), jnp.float32),
        ],
    )(x, ns, gate_w, gb, mlp1_weight, mlp1_bias.reshape(E, 1, 2 * I),
      mlp2_weight, mlp2_bias.reshape(E, 1, H))


# speedup vs baseline: 9.4612x; 3.2319x over previous
import functools
import jax
import jax.numpy as jnp
from jax import lax
from jax.experimental import pallas as pl
from jax.experimental.pallas import tpu as pltpu

H = 1024
I = 1024
E = 8
K = 2
B = 32
LIMIT = 7.0
EPS = 1e-5


def _moe_kernel(x_ref, ns_ref, gw_ref, gb_ref, w1_ref, b1_ref, w2_ref, b2_ref,
                out_ref, t_s, r_s, acc_s, sel_s):
    e = pl.program_id(0)

    @pl.when(e == 0)
    def _routing():
        ir = lax.broadcasted_iota(jnp.int32, (2 * I, 2 * I), 0)
        ic = lax.broadcasted_iota(jnp.int32, (2 * I, 2 * I), 1)
        tgt = 2 * (ic & (I - 1)) + (ic >> 10)
        sel_s[...] = (ir == tgt).astype(jnp.bfloat16)
        xf = x_ref[...].astype(jnp.float32)
        ms = jnp.mean(xf * xf, axis=1, keepdims=True)
        t = xf * lax.rsqrt(ms + EPS) * ns_ref[...]
        t_bf = t.astype(jnp.bfloat16)
        t_s[...] = t_bf
        g = lax.dot_general(t_bf, gw_ref[...], (((1,), (1,)), ((), ())),
                            preferred_element_type=jnp.float32)
        gf = g + gb_ref[...].astype(jnp.float32)
        ids = lax.broadcasted_iota(jnp.int32, (B, E), 1)
        m1 = jnp.max(gf, axis=1, keepdims=True)
        i1 = jnp.min(jnp.where(gf == m1, ids, E), axis=1, keepdims=True)
        g2 = jnp.where(ids == i1, -jnp.inf, gf)
        m2 = jnp.max(g2, axis=1, keepdims=True)
        i2 = jnp.min(jnp.where(g2 == m2, ids, E), axis=1, keepdims=True)
        m1 = m1.astype(jnp.bfloat16).astype(jnp.float32)
        m2 = m2.astype(jnp.bfloat16).astype(jnp.float32)
        d = jnp.exp(m2 - m1)
        p1 = 1.0 / (1.0 + d)
        p2 = d / (1.0 + d)
        r_s[:, 0:1] = i1.astype(jnp.float32)
        r_s[:, 1:2] = i2.astype(jnp.float32)
        r_s[:, 2:3] = p1
        r_s[:, 3:4] = p2

    t_bf = t_s[...]
    w1 = w1_ref[0]
    h = lax.dot_general(t_bf, w1, (((1,), (1,)), ((), ())),
                        preferred_element_type=jnp.float32)
    h_bf = h.astype(jnp.bfloat16) + b1_ref[0]
    hs = lax.dot_general(h_bf, sel_s[...], (((1,), (0,)), ((), ())),
                         preferred_element_type=jnp.float32)
    x_glu = hs[:, :I]
    x_lin = hs[:, I:]
    x_glu = jnp.minimum(x_glu, LIMIT)
    x_lin = jnp.clip(x_lin, -LIMIT, LIMIT)
    out_glu = x_glu * (1.0 / (1.0 + jnp.exp(-1.702 * x_glu)))
    hmid = (out_glu * (x_lin + 1.0)).astype(jnp.bfloat16)

    w2 = w2_ref[0]
    y = lax.dot_general(hmid, w2, (((1,), (1,)), ((), ())),
                        preferred_element_type=jnp.float32)
    y_bf = y.astype(jnp.bfloat16) + b2_ref[0]

    ef = e.astype(jnp.float32)
    w_e = (jnp.where(r_s[:, 0:1] == ef, r_s[:, 2:3], 0.0)
           + jnp.where(r_s[:, 1:2] == ef, r_s[:, 3:4], 0.0))
    contrib = w_e * y_bf.astype(jnp.float32)

    @pl.when(e == 0)
    def _init():
        acc_s[...] = contrib

    @pl.when(e > 0)
    def _accum():
        acc_s[...] += contrib

    @pl.when(e == E - 1)
    def _finish():
        out_ref[...] = (x_ref[...].astype(jnp.float32) + acc_s[...]).astype(
            jnp.bfloat16)


@jax.jit
def kernel(x, norm_scale, gate_w, gate_b, mlp1_weight, mlp1_bias, mlp2_weight,
           mlp2_bias):
    ns = norm_scale.reshape(1, H).astype(jnp.float32)
    gb = gate_b.reshape(1, E)
    return pl.pallas_call(
        _moe_kernel,
        grid=(E,),
        in_specs=[
            pl.BlockSpec((B, H), lambda e: (0, 0)),
            pl.BlockSpec((1, H), lambda e: (0, 0)),
            pl.BlockSpec((E, H), lambda e: (0, 0)),
            pl.BlockSpec((1, E), lambda e: (0, 0)),
            pl.BlockSpec((1, 2 * I, H), lambda e: (e, 0, 0)),
            pl.BlockSpec((1, 1, 2 * I), lambda e: (e, 0, 0)),
            pl.BlockSpec((1, H, I), lambda e: (e, 0, 0)),
            pl.BlockSpec((1, 1, H), lambda e: (e, 0, 0)),
        ],
        out_specs=pl.BlockSpec((B, H), lambda e: (0, 0)),
        out_shape=jax.ShapeDtypeStruct((B, H), jnp.bfloat16),
        scratch_shapes=[
            pltpu.VMEM((B, H), jnp.bfloat16),
            pltpu.VMEM((B, 4), jnp.float32),
            pltpu.VMEM((B, H), jnp.float32),
            pltpu.VMEM((2 * I, 2 * I), jnp.bfloat16),
        ],
    )(x, ns, gate_w, gb, mlp1_weight, mlp1_bias.reshape(E, 1, 2 * I),
      mlp2_weight, mlp2_bias.reshape(E, 1, H))


# speedup vs baseline: 10.1333x; 1.0710x over previous
import jax
import jax.numpy as jnp
from jax import lax
from jax.experimental import pallas as pl
from jax.experimental.pallas import tpu as pltpu

H = 1024
I = 1024
E = 8
K = 2
B = 32
C = 2
CI = 2 * I // C
CO = CI // 2
LIMIT = 7.0
EPS = 1e-5


def _moe_kernel(x_ref, ns_ref, gw_ref, gb_ref, w1_ref, b1_ref, w2_ref, b2_ref,
                out_ref, t_s, r_s, acc_s, y_s, sel_s):
    e = pl.program_id(0)
    j = pl.program_id(1)

    @pl.when((e == 0) & (j == 0))
    def _routing():
        ir = lax.broadcasted_iota(jnp.int32, (CI, CI), 0)
        ic = lax.broadcasted_iota(jnp.int32, (CI, CI), 1)
        tgt = 2 * (ic & (CO - 1)) + (ic // CO)
        sel_s[...] = (ir == tgt).astype(jnp.bfloat16)

        xf = x_ref[...].astype(jnp.float32)
        ms = jnp.mean(xf * xf, axis=1, keepdims=True)
        t = xf * lax.rsqrt(ms + EPS) * ns_ref[...]
        t_bf = t.astype(jnp.bfloat16)
        t_s[...] = t_bf
        g = lax.dot_general(t_bf, gw_ref[...], (((1,), (1,)), ((), ())),
                            preferred_element_type=jnp.float32)
        gf = g + gb_ref[...].astype(jnp.float32)
        ids = lax.broadcasted_iota(jnp.int32, (B, E), 1)
        m1 = jnp.max(gf, axis=1, keepdims=True)
        i1 = jnp.min(jnp.where(gf == m1, ids, E), axis=1, keepdims=True)
        g2 = jnp.where(ids == i1, -jnp.inf, gf)
        m2 = jnp.max(g2, axis=1, keepdims=True)
        i2 = jnp.min(jnp.where(g2 == m2, ids, E), axis=1, keepdims=True)
        m1 = m1.astype(jnp.bfloat16).astype(jnp.float32)
        m2 = m2.astype(jnp.bfloat16).astype(jnp.float32)
        d = jnp.exp(m2 - m1)
        p1 = 1.0 / (1.0 + d)
        p2 = d / (1.0 + d)
        r_s[:, 0:1] = i1.astype(jnp.float32)
        r_s[:, 1:2] = i2.astype(jnp.float32)
        r_s[:, 2:3] = p1
        r_s[:, 3:4] = p2

    t_bf = t_s[...]
    w1 = w1_ref[0]
    h = lax.dot_general(t_bf, w1, (((1,), (1,)), ((), ())),
                        preferred_element_type=jnp.float32)
    h_bf = h.astype(jnp.bfloat16) + b1_ref[0]
    hs = lax.dot_general(h_bf, sel_s[...], (((1,), (0,)), ((), ())),
                         preferred_element_type=jnp.float32)
    x_glu = hs[:, :CO]
    x_lin = hs[:, CO:]
    x_glu = jnp.minimum(x_glu, LIMIT)
    x_lin = jnp.clip(x_lin, -LIMIT, LIMIT)
    out_glu = x_glu * (1.0 / (1.0 + jnp.exp(-1.702 * x_glu)))
    hmid = (out_glu * (x_lin + 1.0)).astype(jnp.bfloat16)

    w2 = w2_ref[0]
    y_part = lax.dot_general(hmid, w2, (((1,), (1,)), ((), ())),
                             preferred_element_type=jnp.float32)

    @pl.when(j == 0)
    def _y_init():
        y_s[...] = y_part

    @pl.when(j == C - 1)
    def _combine():
        y = y_s[...] + y_part if C > 1 else y_part
        y_bf = y.astype(jnp.bfloat16) + b2_ref[0]
        ef = e.astype(jnp.float32)
        w_e = (jnp.where(r_s[:, 0:1] == ef, r_s[:, 2:3], 0.0)
               + jnp.where(r_s[:, 1:2] == ef, r_s[:, 3:4], 0.0))
        contrib = w_e * y_bf.astype(jnp.float32)

        @pl.when(e == 0)
        def _init():
            acc_s[...] = contrib

        @pl.when(e > 0)
        def _accum():
            acc_s[...] += contrib

        @pl.when(e == E - 1)
        def _finish():
            out_ref[...] = (x_ref[...].astype(jnp.float32)
                            + acc_s[...]).astype(jnp.bfloat16)


@jax.jit
def kernel(x, norm_scale, gate_w, gate_b, mlp1_weight, mlp1_bias, mlp2_weight,
           mlp2_bias):
    ns = norm_scale.reshape(1, H).astype(jnp.float32)
    gb = gate_b.reshape(1, E)
    return pl.pallas_call(
        _moe_kernel,
        grid=(E, C),
        in_specs=[
            pl.BlockSpec((B, H), lambda e, j: (0, 0)),
            pl.BlockSpec((1, H), lambda e, j: (0, 0)),
            pl.BlockSpec((E, H), lambda e, j: (0, 0)),
            pl.BlockSpec((1, E), lambda e, j: (0, 0)),
            pl.BlockSpec((1, CI, H), lambda e, j: (e, j, 0)),
            pl.BlockSpec((1, 1, CI), lambda e, j: (e, 0, j)),
            pl.BlockSpec((1, H, CO), lambda e, j: (e, 0, j)),
            pl.BlockSpec((1, 1, H), lambda e, j: (e, 0, 0)),
        ],
        out_specs=pl.BlockSpec((B, H), lambda e, j: (0, 0)),
        out_shape=jax.ShapeDtypeStruct((B, H), jnp.bfloat16),
        scratch_shapes=[
            pltpu.VMEM((B, H), jnp.bfloat16),
            pltpu.VMEM((B, 4), jnp.float32),
            pltpu.VMEM((B, H), jnp.float32),
            pltpu.VMEM((B, H), jnp.float32),
            pltpu.VMEM((CI, CI), jnp.bfloat16),
        ],
    )(x, ns, gate_w, gb, mlp1_weight, mlp1_bias.reshape(E, 1, 2 * I),
      mlp2_weight, mlp2_bias.reshape(E, 1, H))
